# baseline (speedup 1.0000x reference)
import jax
import jax.numpy as jnp
from jax import lax
from jax.experimental import pallas as pl
from jax.experimental.pallas import tpu as pltpu
from jax.experimental.pallas import tpu_sc as plsc

N = 10000
DF = 128
DH = 256
HALF = DH // 2
NT = 16
CHUNK = 80
NCHUNKS = 260
NBUF = 4
NFL = NBUF - 1
EPT = NCHUNKS * CHUNK
EPR = 20625
ACCR = 10240
RPT = ACCR // NT
SHIFT = 14
MASK = (1 << SHIFT) - 1
R = 1000
NB = N // R


def _deg_body(pk_hbm, deg_hbm, pkv, histv, combv, outv, parts_sh):
    c = lax.axis_index("c")
    s = lax.axis_index("s")
    zero16 = jnp.zeros((16,), jnp.float32)
    ones16 = jnp.ones((16,), jnp.float32)

    @pl.when(c == 0)
    def _hist():
        def _z(i, _):
            histv[pl.ds(i * 16, 16)] = zero16
            return 0
        lax.fori_loop(0, ACCR // 16, _z, 0, unroll=8)
        pltpu.sync_copy(pk_hbm.at[pl.ds(s * EPT, EPT)], pkv)

        def _h(i, _):
            pk = pkv[pl.ds(i * 16, 16)]
            idx = lax.shift_right_logical(pk, SHIFT)
            plsc.addupdate_scatter(histv, [idx], ones16)
            return 0
        lax.fori_loop(0, EPT // 16, _h, 0, unroll=8)
        pltpu.sync_copy(histv, parts_sh.at[s])

    plsc.subcore_barrier()

    @pl.when(c == 0)
    def _combine():
        for t in range(NT):
            pltpu.sync_copy(parts_sh.at[t, pl.ds(s * RPT, RPT)], combv.at[t])

        def _c(j, _):
            v = combv[0, pl.ds(j * 16, 16)]
            for t in range(1, NT):
                v = v + combv[t, pl.ds(j * 16, 16)]
            outv[pl.ds(j * 16, 16)] = v
            return 0
        lax.fori_loop(0, RPT // 16, _c, 0)
        pltpu.sync_copy(outv, deg_hbm.at[pl.ds(s * RPT, RPT)])


_deg_call = pl.kernel(
    _deg_body,
    out_type=jax.ShapeDtypeStruct((ACCR,), jnp.float32),
    mesh=plsc.VectorSubcoreMesh(core_axis_name="c", subcore_axis_name="s"),
    scratch_types=[
        pltpu.VMEM((EPT,), jnp.int32),
        pltpu.VMEM((ACCR,), jnp.float32),
        pltpu.VMEM((NT, RPT), jnp.float32),
        pltpu.VMEM((RPT,), jnp.float32),
        pltpu.VMEM_SHARED((NT, ACCR), jnp.float32),
    ],
    compiler_params=pltpu.CompilerParams(needs_layout_passes=False),
)


def _agg_body(h0, h1, pk_hbm, o0, o1, pkb, sidx, didx, rows, acc_sh, gsem, psem):
    c = lax.axis_index("c")
    s = lax.axis_index("s")
    zero16 = jnp.zeros((16,), jnp.float32)

    def _z(i, _):
        rows[0, i // 8, pl.ds((i % 8) * 16, 16)] = zero16
        return 0
    lax.fori_loop(0, CHUNK * 8, _z, 0, unroll=8)
    for k in range(RPT // CHUNK):
        pltpu.sync_copy(rows.at[0], acc_sh.at[pl.ds(s * RPT + k * CHUNK, CHUNK)])
    plsc.subcore_barrier()

    def _run(table, out):
        base = s * EPT

        def _pk_load(j, b):
            pltpu.async_copy(pk_hbm.at[pl.ds(base + j * CHUNK, CHUNK)],
                             pkb.at[b], psem)

        def _pk_wait(j, b):
            pltpu.make_async_copy(pk_hbm.at[pl.ds(base + j * CHUNK, CHUNK)],
                                  pkb.at[b], psem).wait()

        def _unpack(b):
            for k in range(CHUNK // 16):
                pk = pkb[b, pl.ds(k * 16, 16)]
                sidx[b, pl.ds(k * 16, 16)] = jnp.bitwise_and(pk, MASK)
                didx[b, pl.ds(k * 16, 16)] = lax.shift_right_logical(pk, SHIFT)

        def _issue(b):
            pltpu.async_copy(table.at[sidx.at[b]], rows.at[b], gsem)

        def _gwait(b):
            pltpu.make_async_copy(table.at[sidx.at[b]], rows.at[b], gsem).wait()

        for p in range(NBUF):
            _pk_load(p, p)
        for p in range(NFL):
            _pk_wait(p, p)
            _unpack(p)
            _issue(p)
            _pk_load(p + NBUF, p)

        def _outer(t, _):
            jo = t * NBUF
            for b in range(NBUF):
                j = jo + b
                _gwait(b)

                @pl.when(j + NFL < NCHUNKS)
                def _():
                    _pk_wait(j + NFL, (b + NFL) % NBUF)
                    _unpack((b + NFL) % NBUF)
                    _issue((b + NFL) % NBUF)

                    @pl.when(j + NFL + NBUF < NCHUNKS)
                    def _():
                        _pk_load(j + NFL + NBUF, (b + NFL) % NBUF)

                pltpu.sync_copy(rows.at[b], acc_sh.at[didx.at[b]], add=True)
            return 0
        lax.fori_loop(0, NCHUNKS // NBUF, _outer, 0)

        plsc.subcore_barrier()
        for k in range(RPT // CHUNK):
            pltpu.sync_copy(acc_sh.at[pl.ds(s * RPT + k * CHUNK, CHUNK)],
                            out.at[pl.ds(s * RPT + k * CHUNK, CHUNK)])

    @pl.when(c == 0)
    def _():
        _run(h0, o0)

    @pl.when(c == 1)
    def _():
        _run(h1, o1)


_agg_call = pl.kernel(
    _agg_body,
    out_type=[jax.ShapeDtypeStruct((ACCR, HALF), jnp.float32)] * 2,
    mesh=plsc.VectorSubcoreMesh(core_axis_name="c", subcore_axis_name="s"),
    scratch_types=[
        pltpu.VMEM((NBUF, CHUNK), jnp.int32),
        pltpu.VMEM((NBUF, CHUNK), jnp.int32),
        pltpu.VMEM((NBUF, CHUNK), jnp.int32),
        pltpu.VMEM((NBUF, CHUNK, HALF), jnp.float32),
        pltpu.VMEM_SHARED((ACCR, HALF), jnp.float32),
        pltpu.SemaphoreType.DMA,
        pltpu.SemaphoreType.DMA,
    ],
    compiler_params=pltpu.CompilerParams(needs_layout_passes=False),
)


def _tc1_body(deg_ref, x_ref, w_ref, h0_ref, h1_ref):
    dinv = lax.rsqrt(jnp.maximum(deg_ref[...], 1.0))
    h = jnp.dot(x_ref[...], w_ref[...], preferred_element_type=jnp.float32) * dinv
    h0_ref[...] = h[:, :HALF]
    h1_ref[...] = h[:, HALF:]


_tc1_call = pl.pallas_call(
    _tc1_body,
    grid=(NB,),
    in_specs=[
        pl.BlockSpec((R, 1), lambda i: (i, 0)),
        pl.BlockSpec((R, DF), lambda i: (i, 0)),
        pl.BlockSpec((DF, DH), lambda i: (0, 0)),
    ],
    out_specs=[
        pl.BlockSpec((R, HALF), lambda i: (i, 0)),
        pl.BlockSpec((R, HALF), lambda i: (i, 0)),
    ],
    out_shape=[jax.ShapeDtypeStruct((N, HALF), jnp.float32)] * 2,
)


def _tcmid_body(deg_ref, a0_ref, a1_ref, b_ref, g_ref, be_ref, w_ref,
                o_ref, h0_ref, h1_ref, stats):
    ph = pl.program_id(0)
    i = pl.program_id(1)
    dinv = lax.rsqrt(jnp.maximum(deg_ref[...], 1.0))
    x = jnp.concatenate([a0_ref[...], a1_ref[...]], axis=1) * dinv + b_ref[...]

    @pl.when(ph == 0)
    def _():
        @pl.when(i == 0)
        def _():
            stats[...] = jnp.zeros_like(stats)
        stats[0:1, :] += jnp.sum(x, axis=0, keepdims=True)
        stats[1:2, :] += jnp.sum(x * x, axis=0, keepdims=True)

    @pl.when(ph == 1)
    def _():
        m = stats[0:1, :] * (1.0 / N)
        v = stats[1:2, :] * (1.0 / N) - m * m
        y = (x - m) * lax.rsqrt(v + 1e-5) * g_ref[...] + be_ref[...]
        o = jnp.where(y > 0, y, jnp.exp(y) - 1.0)
        o_ref[...] = o
        h = jnp.dot(o, w_ref[...], preferred_element_type=jnp.float32) * dinv
        h0_ref[...] = h[:, :HALF]
        h1_ref[...] = h[:, HALF:]


_tcmid_call = pl.pallas_call(
    _tcmid_body,
    grid=(2, NB),
    in_specs=[
        pl.BlockSpec((R, 1), lambda ph, i: (i, 0)),
        pl.BlockSpec((R, HALF), lambda ph, i: (i, 0)),
        pl.BlockSpec((R, HALF), lambda ph, i: (i, 0)),
        pl.BlockSpec((DH,), lambda ph, i: (0,)),
        pl.BlockSpec((DH,), lambda ph, i: (0,)),
        pl.BlockSpec((DH,), lambda ph, i: (0,)),
        pl.BlockSpec((DH, DH), lambda ph, i: (0, 0)),
    ],
    out_specs=[
        pl.BlockSpec((R, DH), lambda ph, i: (i, 0)),
        pl.BlockSpec((R, HALF), lambda ph, i: (i, 0)),
        pl.BlockSpec((R, HALF), lambda ph, i: (i, 0)),
    ],
    out_shape=[
        jax.ShapeDtypeStruct((N, DH), jnp.float32),
        jax.ShapeDtypeStruct((N, HALF), jnp.float32),
        jax.ShapeDtypeStruct((N, HALF), jnp.float32),
    ],
    scratch_shapes=[pltpu.VMEM((2, DH), jnp.float32)],
)


def _tcf_body(deg_ref, a0_ref, a1_ref, b_ref, g_ref, be_ref, jk_ref,
              o1_ref, o2_ref, out_ref, stats):
    ph = pl.program_id(0)
    i = pl.program_id(1)
    dinv = lax.rsqrt(jnp.maximum(deg_ref[...], 1.0))
    x = jnp.concatenate([a0_ref[...], a1_ref[...]], axis=1) * dinv + b_ref[...]

    @pl.when(ph == 0)
    def _():
        @pl.when(i == 0)
        def _():
            stats[...] = jnp.zeros_like(stats)
        stats[0:1, :] += jnp.sum(x, axis=0, keepdims=True)
        stats[1:2, :] += jnp.sum(x * x, axis=0, keepdims=True)

    @pl.when(ph == 1)
    def _():
        m = stats[0:1, :] * (1.0 / N)
        v = stats[1:2, :] * (1.0 / N) - m * m
        y = (x - m) * lax.rsqrt(v + 1e-5) * g_ref[...] + be_ref[...]
        o3 = jnp.where(y > 0, y, jnp.exp(y) - 1.0)
        w0, w1, w2 = jk_ref[0], jk_ref[1], jk_ref[2]
        mx = jnp.maximum(jnp.maximum(w0, w1), w2)
        e0 = jnp.exp(w0 - mx)
        e1 = jnp.exp(w1 - mx)
        e2 = jnp.exp(w2 - mx)
        inv = 1.0 / (e0 + e1 + e2)
        out_ref[...] = ((e0 * inv) * o1_ref[...] + (e1 * inv) * o2_ref[...]
                        + (e2 * inv) * o3)


_tcf_call = pl.pallas_call(
    _tcf_body,
    grid=(2, NB),
    in_specs=[
        pl.BlockSpec((R, 1), lambda ph, i: (i, 0)),
        pl.BlockSpec((R, HALF), lambda ph, i: (i, 0)),
        pl.BlockSpec((R, HALF), lambda ph, i: (i, 0)),
        pl.BlockSpec((DH,), lambda ph, i: (0,)),
        pl.BlockSpec((DH,), lambda ph, i: (0,)),
        pl.BlockSpec((DH,), lambda ph, i: (0,)),
        pl.BlockSpec(memory_space=pltpu.SMEM),
        pl.BlockSpec((R, DH), lambda ph, i: (i, 0)),
        pl.BlockSpec((R, DH), lambda ph, i: (i, 0)),
    ],
    out_specs=pl.BlockSpec((R, DH), lambda ph, i: (i, 0)),
    out_shape=jax.ShapeDtypeStruct((N, DH), jnp.float32),
    scratch_shapes=[pltpu.VMEM((2, DH), jnp.float32)],
)


def kernel(adj, features, W1, b1, W2, b2, W3, b3, gamma1, beta1, gamma2,
           beta2, gamma3, beta3, jk_weights):
    loop = jnp.arange(N, dtype=jnp.int32)
    src = jnp.concatenate([adj[0].astype(jnp.int32), loop])
    dst = jnp.concatenate([adj[1].astype(jnp.int32), loop])
    pk = (src + dst * (1 << SHIFT)).reshape(NT, EPR)
    ndum = EPT - EPR
    dumdst = N + (jnp.arange(NT * ndum, dtype=jnp.int32) % (ACCR - N))
    dum = (dumdst * (1 << SHIFT)).reshape(NT, ndum)
    pk3 = jnp.concatenate([pk, dum], axis=1).reshape(NT * EPT)

    deg = _deg_call(pk3)
    degc = deg[:N].reshape(N, 1)

    h0, h1 = _tc1_call(degc, features, W1)
    a0, a1 = _agg_call(h0, h1, pk3)
    o1, h0, h1 = _tcmid_call(degc, a0, a1, b1, gamma1, beta1, W2)
    a0, a1 = _agg_call(h0, h1, pk3)
    o2, h0, h1 = _tcmid_call(degc, a0, a1, b2, gamma2, beta2, W3)
    a0, a1 = _agg_call(h0, h1, pk3)
    return _tcf_call(degc, a0, a1, b3, gamma3, beta3, jk_weights, o1, o2)


# speedup vs baseline: 1.1334x; 1.1334x over previous
import jax
import jax.numpy as jnp
from jax import lax
from jax.experimental import pallas as pl
from jax.experimental.pallas import tpu as pltpu
from jax.experimental.pallas import tpu_sc as plsc

N = 10000
DF = 128
DH = 256
HALF = DH // 2
NT = 16
CHUNK = 96
NCHUNKS = 216
NBUF = 3
NFL = NBUF - 1
NFULL = 6
NREM = 64
EPT = NCHUNKS * CHUNK
EPR = 20625
ACCR = 10240
RPT = ACCR // NT
SHIFT = 14
MASK = (1 << SHIFT) - 1
R = 1000
NB = N // R


def _deg_body(pk_hbm, deg_hbm, pkv, histv, combv, outv, parts_sh):
    c = lax.axis_index("c")
    s = lax.axis_index("s")
    zero16 = jnp.zeros((16,), jnp.float32)
    ones16 = jnp.ones((16,), jnp.float32)

    @pl.when(c == 0)
    def _hist():
        def _z(i, _):
            histv[pl.ds(i * 16, 16)] = zero16
            return 0
        lax.fori_loop(0, ACCR // 16, _z, 0, unroll=8)
        pltpu.sync_copy(pk_hbm.at[pl.ds(s * EPT, EPT)], pkv)

        def _h(i, _):
            pk = pkv[pl.ds(i * 16, 16)]
            idx = lax.shift_right_logical(pk, SHIFT)
            plsc.addupdate_scatter(histv, [idx], ones16)
            return 0
        lax.fori_loop(0, EPT // 16, _h, 0, unroll=8)
        pltpu.sync_copy(histv, parts_sh.at[s])

    plsc.subcore_barrier()

    @pl.when(c == 0)
    def _combine():
        for t in range(NT):
            pltpu.sync_copy(parts_sh.at[t, pl.ds(s * RPT, RPT)], combv.at[t])

        def _c(j, _):
            v = combv[0, pl.ds(j * 16, 16)]
            for t in range(1, NT):
                v = v + combv[t, pl.ds(j * 16, 16)]
            outv[pl.ds(j * 16, 16)] = v
            return 0
        lax.fori_loop(0, RPT // 16, _c, 0)
        pltpu.sync_copy(outv, deg_hbm.at[pl.ds(s * RPT, RPT)])


_deg_call = pl.kernel(
    _deg_body,
    out_type=jax.ShapeDtypeStruct((ACCR,), jnp.float32),
    mesh=plsc.VectorSubcoreMesh(core_axis_name="c", subcore_axis_name="s"),
    scratch_types=[
        pltpu.VMEM((EPT,), jnp.int32),
        pltpu.VMEM((ACCR,), jnp.float32),
        pltpu.VMEM((NT, RPT), jnp.float32),
        pltpu.VMEM((RPT,), jnp.float32),
        pltpu.VMEM_SHARED((NT, ACCR), jnp.float32),
    ],
    compiler_params=pltpu.CompilerParams(needs_layout_passes=False),
)


def _agg_body(h0, h1, pk_hbm, o0, o1, pkb, sidx, didx, rows, acc_sh, gsem, psem):
    c = lax.axis_index("c")
    s = lax.axis_index("s")
    zero16 = jnp.zeros((16,), jnp.float32)

    def _z(i, _):
        rows[0, i // 8, pl.ds((i % 8) * 16, 16)] = zero16
        return 0
    lax.fori_loop(0, CHUNK * 8, _z, 0, unroll=8)
    for k in range(NFULL):
        pltpu.sync_copy(rows.at[0], acc_sh.at[pl.ds(s * RPT + k * CHUNK, CHUNK)])
    pltpu.sync_copy(rows.at[0, pl.ds(0, NREM)],
                    acc_sh.at[pl.ds(s * RPT + NFULL * CHUNK, NREM)])
    plsc.subcore_barrier()

    def _run(table, out):
        base = s * EPT

        def _pk_load(j, b):
            pltpu.async_copy(pk_hbm.at[pl.ds(base + j * CHUNK, CHUNK)],
                             pkb.at[b], psem)

        def _pk_wait(j, b):
            pltpu.make_async_copy(pk_hbm.at[pl.ds(base + j * CHUNK, CHUNK)],
                                  pkb.at[b], psem).wait()

        def _unpack(b):
            for k in range(CHUNK // 16):
                pk = pkb[b, pl.ds(k * 16, 16)]
                sidx[b, pl.ds(k * 16, 16)] = jnp.bitwise_and(pk, MASK)
                didx[b, pl.ds(k * 16, 16)] = lax.shift_right_logical(pk, SHIFT)

        def _issue(b):
            pltpu.async_copy(table.at[sidx.at[b]], rows.at[b], gsem)

        def _gwait(b):
            pltpu.make_async_copy(table.at[sidx.at[b]], rows.at[b], gsem).wait()

        for p in range(NBUF):
            _pk_load(p, p)
        for p in range(NFL):
            _pk_wait(p, p)
            _unpack(p)
            _issue(p)
            _pk_load(p + NBUF, p)

        def _outer(t, _):
            jo = t * NBUF
            for b in range(NBUF):
                j = jo + b
                _gwait(b)

                @pl.when(j + NFL < NCHUNKS)
                def _():
                    _pk_wait(j + NFL, (b + NFL) % NBUF)
                    _unpack((b + NFL) % NBUF)
                    _issue((b + NFL) % NBUF)

                    @pl.when(j + NFL + NBUF < NCHUNKS)
                    def _():
                        _pk_load(j + NFL + NBUF, (b + NFL) % NBUF)

                pltpu.sync_copy(rows.at[b], acc_sh.at[didx.at[b]], add=True)
            return 0
        lax.fori_loop(0, NCHUNKS // NBUF, _outer, 0)

        plsc.subcore_barrier()
        for k in range(NFULL):
            pltpu.sync_copy(acc_sh.at[pl.ds(s * RPT + k * CHUNK, CHUNK)],
                            out.at[pl.ds(s * RPT + k * CHUNK, CHUNK)])
        pltpu.sync_copy(acc_sh.at[pl.ds(s * RPT + NFULL * CHUNK, NREM)],
                        out.at[pl.ds(s * RPT + NFULL * CHUNK, NREM)])

    @pl.when(c == 0)
    def _():
        _run(h0, o0)

    @pl.when(c == 1)
    def _():
        _run(h1, o1)


_agg_call = pl.kernel(
    _agg_body,
    out_type=[jax.ShapeDtypeStruct((ACCR, HALF), jnp.float32)] * 2,
    mesh=plsc.VectorSubcoreMesh(core_axis_name="c", subcore_axis_name="s"),
    scratch_types=[
        pltpu.VMEM((NBUF, CHUNK), jnp.int32),
        pltpu.VMEM((NBUF, CHUNK), jnp.int32),
        pltpu.VMEM((NBUF, CHUNK), jnp.int32),
        pltpu.VMEM((NBUF, CHUNK, HALF), jnp.float32),
        pltpu.VMEM_SHARED((ACCR, HALF), jnp.float32),
        pltpu.SemaphoreType.DMA,
        pltpu.SemaphoreType.DMA,
    ],
    compiler_params=pltpu.CompilerParams(needs_layout_passes=False),
)


def _tc1_body(deg_ref, x_ref, w_ref, h0_ref, h1_ref):
    dinv = lax.rsqrt(jnp.maximum(deg_ref[...], 1.0))
    h = jnp.dot(x_ref[...], w_ref[...], preferred_element_type=jnp.float32) * dinv
    h0_ref[...] = h[:, :HALF]
    h1_ref[...] = h[:, HALF:]


_tc1_call = pl.pallas_call(
    _tc1_body,
    grid=(NB,),
    in_specs=[
        pl.BlockSpec((R, 1), lambda i: (i, 0)),
        pl.BlockSpec((R, DF), lambda i: (i, 0)),
        pl.BlockSpec((DF, DH), lambda i: (0, 0)),
    ],
    out_specs=[
        pl.BlockSpec((R, HALF), lambda i: (i, 0)),
        pl.BlockSpec((R, HALF), lambda i: (i, 0)),
    ],
    out_shape=[jax.ShapeDtypeStruct((N, HALF), jnp.float32)] * 2,
)


def _tcmid_body(deg_ref, a0_ref, a1_ref, b_ref, g_ref, be_ref, w_ref,
                o_ref, h0_ref, h1_ref, stats):
    ph = pl.program_id(0)
    i = pl.program_id(1)
    dinv = lax.rsqrt(jnp.maximum(deg_ref[...], 1.0))
    x = jnp.concatenate([a0_ref[...], a1_ref[...]], axis=1) * dinv + b_ref[...]

    @pl.when(ph == 0)
    def _():
        @pl.when(i == 0)
        def _():
            stats[...] = jnp.zeros_like(stats)
        stats[0:1, :] += jnp.sum(x, axis=0, keepdims=True)
        stats[1:2, :] += jnp.sum(x * x, axis=0, keepdims=True)

    @pl.when(ph == 1)
    def _():
        m = stats[0:1, :] * (1.0 / N)
        v = stats[1:2, :] * (1.0 / N) - m * m
        y = (x - m) * lax.rsqrt(v + 1e-5) * g_ref[...] + be_ref[...]
        o = jnp.where(y > 0, y, jnp.exp(y) - 1.0)
        o_ref[...] = o
        h = jnp.dot(o, w_ref[...], preferred_element_type=jnp.float32) * dinv
        h0_ref[...] = h[:, :HALF]
        h1_ref[...] = h[:, HALF:]


_tcmid_call = pl.pallas_call(
    _tcmid_body,
    grid=(2, NB),
    in_specs=[
        pl.BlockSpec((R, 1), lambda ph, i: (i, 0)),
        pl.BlockSpec((R, HALF), lambda ph, i: (i, 0)),
        pl.BlockSpec((R, HALF), lambda ph, i: (i, 0)),
        pl.BlockSpec((DH,), lambda ph, i: (0,)),
        pl.BlockSpec((DH,), lambda ph, i: (0,)),
        pl.BlockSpec((DH,), lambda ph, i: (0,)),
        pl.BlockSpec((DH, DH), lambda ph, i: (0, 0)),
    ],
    out_specs=[
        pl.BlockSpec((R, DH), lambda ph, i: (i, 0)),
        pl.BlockSpec((R, HALF), lambda ph, i: (i, 0)),
        pl.BlockSpec((R, HALF), lambda ph, i: (i, 0)),
    ],
    out_shape=[
        jax.ShapeDtypeStruct((N, DH), jnp.float32),
        jax.ShapeDtypeStruct((N, HALF), jnp.float32),
        jax.ShapeDtypeStruct((N, HALF), jnp.float32),
    ],
    scratch_shapes=[pltpu.VMEM((2, DH), jnp.float32)],
)


def _tcf_body(deg_ref, a0_ref, a1_ref, b_ref, g_ref, be_ref, jk_ref,
              o1_ref, o2_ref, out_ref, stats):
    ph = pl.program_id(0)
    i = pl.program_id(1)
    dinv = lax.rsqrt(jnp.maximum(deg_ref[...], 1.0))
    x = jnp.concatenate([a0_ref[...], a1_ref[...]], axis=1) * dinv + b_ref[...]

    @pl.when(ph == 0)
    def _():
        @pl.when(i == 0)
        def _():
            stats[...] = jnp.zeros_like(stats)
        stats[0:1, :] += jnp.sum(x, axis=0, keepdims=True)
        stats[1:2, :] += jnp.sum(x * x, axis=0, keepdims=True)

    @pl.when(ph == 1)
    def _():
        m = stats[0:1, :] * (1.0 / N)
        v = stats[1:2, :] * (1.0 / N) - m * m
        y = (x - m) * lax.rsqrt(v + 1e-5) * g_ref[...] + be_ref[...]
        o3 = jnp.where(y > 0, y, jnp.exp(y) - 1.0)
        w0, w1, w2 = jk_ref[0], jk_ref[1], jk_ref[2]
        mx = jnp.maximum(jnp.maximum(w0, w1), w2)
        e0 = jnp.exp(w0 - mx)
        e1 = jnp.exp(w1 - mx)
        e2 = jnp.exp(w2 - mx)
        inv = 1.0 / (e0 + e1 + e2)
        out_ref[...] = ((e0 * inv) * o1_ref[...] + (e1 * inv) * o2_ref[...]
                        + (e2 * inv) * o3)


_tcf_call = pl.pallas_call(
    _tcf_body,
    grid=(2, NB),
    in_specs=[
        pl.BlockSpec((R, 1), lambda ph, i: (i, 0)),
        pl.BlockSpec((R, HALF), lambda ph, i: (i, 0)),
        pl.BlockSpec((R, HALF), lambda ph, i: (i, 0)),
        pl.BlockSpec((DH,), lambda ph, i: (0,)),
        pl.BlockSpec((DH,), lambda ph, i: (0,)),
        pl.BlockSpec((DH,), lambda ph, i: (0,)),
        pl.BlockSpec(memory_space=pltpu.SMEM),
        pl.BlockSpec((R, DH), lambda ph, i: (i, 0)),
        pl.BlockSpec((R, DH), lambda ph, i: (i, 0)),
    ],
    out_specs=pl.BlockSpec((R, DH), lambda ph, i: (i, 0)),
    out_shape=jax.ShapeDtypeStruct((N, DH), jnp.float32),
    scratch_shapes=[pltpu.VMEM((2, DH), jnp.float32)],
)


def kernel(adj, features, W1, b1, W2, b2, W3, b3, gamma1, beta1, gamma2,
           beta2, gamma3, beta3, jk_weights):
    loop = jnp.arange(N, dtype=jnp.int32)
    src = jnp.concatenate([adj[0].astype(jnp.int32), loop])
    dst = jnp.concatenate([adj[1].astype(jnp.int32), loop])
    pk = (src + dst * (1 << SHIFT)).reshape(NT, EPR)
    ndum = EPT - EPR
    dumdst = N + (jnp.arange(NT * ndum, dtype=jnp.int32) % (ACCR - N))
    dum = (dumdst * (1 << SHIFT)).reshape(NT, ndum)
    pk3 = jnp.concatenate([pk, dum], axis=1).reshape(NT * EPT)

    deg = _deg_call(pk3)
    degc = deg[:N].reshape(N, 1)

    h0, h1 = _tc1_call(degc, features, W1)
    a0, a1 = _agg_call(h0, h1, pk3)
    o1, h0, h1 = _tcmid_call(degc, a0, a1, b1, gamma1, beta1, W2)
    a0, a1 = _agg_call(h0, h1, pk3)
    o2, h0, h1 = _tcmid_call(degc, a0, a1, b2, gamma2, beta2, W3)
    a0, a1 = _agg_call(h0, h1, pk3)
    return _tcf_call(degc, a0, a1, b3, gamma3, beta3, jk_weights, o1, o2)


# speedup vs baseline: 1.1544x; 1.0185x over previous
import jax
import jax.numpy as jnp
from jax import lax
from jax.experimental import pallas as pl
from jax.experimental.pallas import tpu as pltpu
from jax.experimental.pallas import tpu_sc as plsc

N = 10000
DF = 128
DH = 256
HALF = DH // 2
NT = 16
CHUNK = 96
NCHUNKS = 216
NBUF = 3
NFL = NBUF - 1
NFULL = 6
NREM = 64
EPT = NCHUNKS * CHUNK
EPR = 20625
ACCR = 10240
RPT = ACCR // NT
SHIFT = 14
MASK = (1 << SHIFT) - 1
R = 2000
NB = N // R


def _deg_body(pk_hbm, deg_hbm, pkv, histv, combv, outv, parts_sh):
    c = lax.axis_index("c")
    s = lax.axis_index("s")
    zero16 = jnp.zeros((16,), jnp.float32)
    ones16 = jnp.ones((16,), jnp.float32)

    @pl.when(c == 0)
    def _hist():
        def _z(i, _):
            histv[pl.ds(i * 16, 16)] = zero16
            return 0
        lax.fori_loop(0, ACCR // 16, _z, 0, unroll=8)
        pltpu.sync_copy(pk_hbm.at[pl.ds(s * EPT, EPT)], pkv)

        def _h(i, _):
            pk = pkv[pl.ds(i * 16, 16)]
            idx = lax.shift_right_logical(pk, SHIFT)
            plsc.addupdate_scatter(histv, [idx], ones16)
            return 0
        lax.fori_loop(0, EPT // 16, _h, 0, unroll=8)
        pltpu.sync_copy(histv, parts_sh.at[s])

    plsc.subcore_barrier()

    @pl.when(c == 0)
    def _combine():
        for t in range(NT):
            pltpu.sync_copy(parts_sh.at[t, pl.ds(s * RPT, RPT)], combv.at[t])

        def _c(j, _):
            v = combv[0, pl.ds(j * 16, 16)]
            for t in range(1, NT):
                v = v + combv[t, pl.ds(j * 16, 16)]
            outv[pl.ds(j * 16, 16)] = v
            return 0
        lax.fori_loop(0, RPT // 16, _c, 0)
        pltpu.sync_copy(outv, deg_hbm.at[pl.ds(s * RPT, RPT)])


_deg_call = pl.kernel(
    _deg_body,
    out_type=jax.ShapeDtypeStruct((ACCR,), jnp.float32),
    mesh=plsc.VectorSubcoreMesh(core_axis_name="c", subcore_axis_name="s"),
    scratch_types=[
        pltpu.VMEM((EPT,), jnp.int32),
        pltpu.VMEM((ACCR,), jnp.float32),
        pltpu.VMEM((NT, RPT), jnp.float32),
        pltpu.VMEM((RPT,), jnp.float32),
        pltpu.VMEM_SHARED((NT, ACCR), jnp.float32),
    ],
    compiler_params=pltpu.CompilerParams(needs_layout_passes=False),
)


def _agg_body(h0, h1, pk_hbm, o0, o1, pkb, sidx, didx, rows, acc_sh, gsem, psem):
    c = lax.axis_index("c")
    s = lax.axis_index("s")
    zero16 = jnp.zeros((16,), jnp.float32)

    def _z(i, _):
        rows[0, i // 8, pl.ds((i % 8) * 16, 16)] = zero16
        return 0
    lax.fori_loop(0, CHUNK * 8, _z, 0, unroll=8)
    for k in range(NFULL):
        pltpu.sync_copy(rows.at[0], acc_sh.at[pl.ds(s * RPT + k * CHUNK, CHUNK)])
    pltpu.sync_copy(rows.at[0, pl.ds(0, NREM)],
                    acc_sh.at[pl.ds(s * RPT + NFULL * CHUNK, NREM)])
    plsc.subcore_barrier()

    def _run(table, out):
        base = s * EPT

        def _pk_load(j, b):
            pltpu.async_copy(pk_hbm.at[pl.ds(base + j * CHUNK, CHUNK)],
                             pkb.at[b], psem)

        def _pk_wait(j, b):
            pltpu.make_async_copy(pk_hbm.at[pl.ds(base + j * CHUNK, CHUNK)],
                                  pkb.at[b], psem).wait()

        def _unpack(b):
            for k in range(CHUNK // 16):
                pk = pkb[b, pl.ds(k * 16, 16)]
                sidx[b, pl.ds(k * 16, 16)] = jnp.bitwise_and(pk, MASK)
                didx[b, pl.ds(k * 16, 16)] = lax.shift_right_logical(pk, SHIFT)

        def _issue(b):
            pltpu.async_copy(table.at[sidx.at[b]], rows.at[b], gsem)

        def _gwait(b):
            pltpu.make_async_copy(table.at[sidx.at[b]], rows.at[b], gsem).wait()

        for p in range(NBUF):
            _pk_load(p, p)
        for p in range(NFL):
            _pk_wait(p, p)
            _unpack(p)
            _issue(p)
            _pk_load(p + NBUF, p)

        def _outer(t, _):
            jo = t * NBUF
            for b in range(NBUF):
                j = jo + b
                _gwait(b)

                @pl.when(j + NFL < NCHUNKS)
                def _():
                    _pk_wait(j + NFL, (b + NFL) % NBUF)
                    _unpack((b + NFL) % NBUF)
                    _issue((b + NFL) % NBUF)

                    @pl.when(j + NFL + NBUF < NCHUNKS)
                    def _():
                        _pk_load(j + NFL + NBUF, (b + NFL) % NBUF)

                pltpu.sync_copy(rows.at[b], acc_sh.at[didx.at[b]], add=True)
            return 0
        lax.fori_loop(0, NCHUNKS // NBUF, _outer, 0)

        plsc.subcore_barrier()
        for k in range(NFULL):
            pltpu.sync_copy(acc_sh.at[pl.ds(s * RPT + k * CHUNK, CHUNK)],
                            out.at[pl.ds(s * RPT + k * CHUNK, CHUNK)])
        pltpu.sync_copy(acc_sh.at[pl.ds(s * RPT + NFULL * CHUNK, NREM)],
                        out.at[pl.ds(s * RPT + NFULL * CHUNK, NREM)])

    @pl.when(c == 0)
    def _():
        _run(h0, o0)

    @pl.when(c == 1)
    def _():
        _run(h1, o1)


_agg_call = pl.kernel(
    _agg_body,
    out_type=[jax.ShapeDtypeStruct((ACCR, HALF), jnp.float32)] * 2,
    mesh=plsc.VectorSubcoreMesh(core_axis_name="c", subcore_axis_name="s"),
    scratch_types=[
        pltpu.VMEM((NBUF, CHUNK), jnp.int32),
        pltpu.VMEM((NBUF, CHUNK), jnp.int32),
        pltpu.VMEM((NBUF, CHUNK), jnp.int32),
        pltpu.VMEM((NBUF, CHUNK, HALF), jnp.float32),
        pltpu.VMEM_SHARED((ACCR, HALF), jnp.float32),
        pltpu.SemaphoreType.DMA,
        pltpu.SemaphoreType.DMA,
    ],
    compiler_params=pltpu.CompilerParams(needs_layout_passes=False),
)


def _tc1_body(deg_ref, x_ref, w_ref, h0_ref, h1_ref):
    dinv = lax.rsqrt(jnp.maximum(deg_ref[...], 1.0))
    h = jnp.dot(x_ref[...], w_ref[...], preferred_element_type=jnp.float32) * dinv
    h0_ref[...] = h[:, :HALF]
    h1_ref[...] = h[:, HALF:]


_tc1_call = pl.pallas_call(
    _tc1_body,
    grid=(NB,),
    in_specs=[
        pl.BlockSpec((R, 1), lambda i: (i, 0)),
        pl.BlockSpec((R, DF), lambda i: (i, 0)),
        pl.BlockSpec((DF, DH), lambda i: (0, 0)),
    ],
    out_specs=[
        pl.BlockSpec((R, HALF), lambda i: (i, 0)),
        pl.BlockSpec((R, HALF), lambda i: (i, 0)),
    ],
    out_shape=[jax.ShapeDtypeStruct((N, HALF), jnp.float32)] * 2,
)


def _tcmid_body(deg_ref, a0_ref, a1_ref, b_ref, g_ref, be_ref, w_ref,
                o_ref, h0_ref, h1_ref, stats):
    ph = pl.program_id(0)
    i = pl.program_id(1)
    dinv = lax.rsqrt(jnp.maximum(deg_ref[...], 1.0))
    x = jnp.concatenate([a0_ref[...], a1_ref[...]], axis=1) * dinv + b_ref[...]

    @pl.when(ph == 0)
    def _():
        @pl.when(i == 0)
        def _():
            stats[...] = jnp.zeros_like(stats)
        stats[0:1, :] += jnp.sum(x, axis=0, keepdims=True)
        stats[1:2, :] += jnp.sum(x * x, axis=0, keepdims=True)

    @pl.when(ph == 1)
    def _():
        m = stats[0:1, :] * (1.0 / N)
        v = stats[1:2, :] * (1.0 / N) - m * m
        y = (x - m) * lax.rsqrt(v + 1e-5) * g_ref[...] + be_ref[...]
        o = jnp.where(y > 0, y, jnp.exp(y) - 1.0)
        o_ref[...] = o
        h = jnp.dot(o, w_ref[...], preferred_element_type=jnp.float32) * dinv
        h0_ref[...] = h[:, :HALF]
        h1_ref[...] = h[:, HALF:]


_tcmid_call = pl.pallas_call(
    _tcmid_body,
    grid=(2, NB),
    in_specs=[
        pl.BlockSpec((R, 1), lambda ph, i: (i, 0)),
        pl.BlockSpec((R, HALF), lambda ph, i: (i, 0)),
        pl.BlockSpec((R, HALF), lambda ph, i: (i, 0)),
        pl.BlockSpec((DH,), lambda ph, i: (0,)),
        pl.BlockSpec((DH,), lambda ph, i: (0,)),
        pl.BlockSpec((DH,), lambda ph, i: (0,)),
        pl.BlockSpec((DH, DH), lambda ph, i: (0, 0)),
    ],
    out_specs=[
        pl.BlockSpec((R, DH), lambda ph, i: (i, 0)),
        pl.BlockSpec((R, HALF), lambda ph, i: (i, 0)),
        pl.BlockSpec((R, HALF), lambda ph, i: (i, 0)),
    ],
    out_shape=[
        jax.ShapeDtypeStruct((N, DH), jnp.float32),
        jax.ShapeDtypeStruct((N, HALF), jnp.float32),
        jax.ShapeDtypeStruct((N, HALF), jnp.float32),
    ],
    scratch_shapes=[pltpu.VMEM((2, DH), jnp.float32)],
)


def _tcf_body(deg_ref, a0_ref, a1_ref, b_ref, g_ref, be_ref, jk_ref,
              o1_ref, o2_ref, out_ref, stats):
    ph = pl.program_id(0)
    i = pl.program_id(1)
    dinv = lax.rsqrt(jnp.maximum(deg_ref[...], 1.0))
    x = jnp.concatenate([a0_ref[...], a1_ref[...]], axis=1) * dinv + b_ref[...]

    @pl.when(ph == 0)
    def _():
        @pl.when(i == 0)
        def _():
            stats[...] = jnp.zeros_like(stats)
        stats[0:1, :] += jnp.sum(x, axis=0, keepdims=True)
        stats[1:2, :] += jnp.sum(x * x, axis=0, keepdims=True)

    @pl.when(ph == 1)
    def _():
        m = stats[0:1, :] * (1.0 / N)
        v = stats[1:2, :] * (1.0 / N) - m * m
        y = (x - m) * lax.rsqrt(v + 1e-5) * g_ref[...] + be_ref[...]
        o3 = jnp.where(y > 0, y, jnp.exp(y) - 1.0)
        w0, w1, w2 = jk_ref[0], jk_ref[1], jk_ref[2]
        mx = jnp.maximum(jnp.maximum(w0, w1), w2)
        e0 = jnp.exp(w0 - mx)
        e1 = jnp.exp(w1 - mx)
        e2 = jnp.exp(w2 - mx)
        inv = 1.0 / (e0 + e1 + e2)
        out_ref[...] = ((e0 * inv) * o1_ref[...] + (e1 * inv) * o2_ref[...]
                        + (e2 * inv) * o3)


_tcf_call = pl.pallas_call(
    _tcf_body,
    grid=(2, NB),
    in_specs=[
        pl.BlockSpec((R, 1), lambda ph, i: (i, 0)),
        pl.BlockSpec((R, HALF), lambda ph, i: (i, 0)),
        pl.BlockSpec((R, HALF), lambda ph, i: (i, 0)),
        pl.BlockSpec((DH,), lambda ph, i: (0,)),
        pl.BlockSpec((DH,), lambda ph, i: (0,)),
        pl.BlockSpec((DH,), lambda ph, i: (0,)),
        pl.BlockSpec(memory_space=pltpu.SMEM),
        pl.BlockSpec((R, DH), lambda ph, i: (i, 0)),
        pl.BlockSpec((R, DH), lambda ph, i: (i, 0)),
    ],
    out_specs=pl.BlockSpec((R, DH), lambda ph, i: (i, 0)),
    out_shape=jax.ShapeDtypeStruct((N, DH), jnp.float32),
    scratch_shapes=[pltpu.VMEM((2, DH), jnp.float32)],
)


def kernel(adj, features, W1, b1, W2, b2, W3, b3, gamma1, beta1, gamma2,
           beta2, gamma3, beta3, jk_weights):
    loop = jnp.arange(N, dtype=jnp.int32)
    src = jnp.concatenate([adj[0].astype(jnp.int32), loop])
    dst = jnp.concatenate([adj[1].astype(jnp.int32), loop])
    pk = (src + dst * (1 << SHIFT)).reshape(NT, EPR)
    ndum = EPT - EPR
    dumdst = N + (jnp.arange(NT * ndum, dtype=jnp.int32) % (ACCR - N))
    dum = (dumdst * (1 << SHIFT)).reshape(NT, ndum)
    pk3 = jnp.concatenate([pk, dum], axis=1).reshape(NT * EPT)

    deg = _deg_call(pk3)
    degc = deg[:N].reshape(N, 1)

    h0, h1 = _tc1_call(degc, features, W1)
    a0, a1 = _agg_call(h0, h1, pk3)
    o1, h0, h1 = _tcmid_call(degc, a0, a1, b1, gamma1, beta1, W2)
    a0, a1 = _agg_call(h0, h1, pk3)
    o2, h0, h1 = _tcmid_call(degc, a0, a1, b2, gamma2, beta2, W3)
    a0, a1 = _agg_call(h0, h1, pk3)
    return _tcf_call(degc, a0, a1, b3, gamma3, beta3, jk_weights, o1, o2)
